# baseline (speedup 1.0000x reference)
import jax
import jax.numpy as jnp
from jax import lax
from jax.experimental import pallas as pl
from jax.experimental.pallas import tpu as pltpu
from jax.experimental.pallas import tpu_sc as plsc

N = 10000
E = 320000
NC = 2
NS = 16
NW = NC * NS
EPT = E // NW
CH = 125
NCHUNK = EPT // CH
NP_ = 10112
NPS = NP_ // NS
LASTROWS = N - (NS - 1) * NPS
DW = 8
NB = 4
NI = 8

_mesh = plsc.VectorSubcoreMesh(core_axis_name="c", subcore_axis_name="s")


def _deg_body(dst3_hbm, ones_hbm, zeros_hbm, deg_hbm, dstbuf, ones_v, acc, sem4):
    c = lax.axis_index("c")
    s = lax.axis_index("s")
    w = c * NS + s

    pltpu.sync_copy(dst3_hbm.at[w], dstbuf)
    pltpu.sync_copy(ones_hbm, ones_v)
    pltpu.sync_copy(zeros_hbm, acc.at[pl.ds(s * NPS, NPS)])
    plsc.subcore_barrier()

    def step(j, carry):
        @pl.when(j >= 4)
        def _():
            pltpu.make_async_copy(ones_v, acc.at[dstbuf.at[j - 4]],
                                  sem4.at[j & 3]).wait()

        pltpu.async_copy(ones_v, acc.at[dstbuf.at[j]], sem4.at[j & 3],
                         add=True)
        return carry

    lax.fori_loop(0, NCHUNK, step, 0)

    for jt in range(NCHUNK - 4, NCHUNK):
        pltpu.make_async_copy(ones_v, acc.at[dstbuf.at[jt]],
                              sem4.at[jt & 3]).wait()

    plsc.subcore_barrier()

    @pl.when(s < NS - 1)
    def _():
        pltpu.sync_copy(acc.at[pl.ds(s * NPS, NPS)],
                        deg_hbm.at[c, pl.ds(s * NPS, NPS)])

    @pl.when(s == NS - 1)
    def _():
        pltpu.sync_copy(acc.at[pl.ds((NS - 1) * NPS, LASTROWS)],
                        deg_hbm.at[c, pl.ds((NS - 1) * NPS, LASTROWS)])


def _sc_degree(dst3, ones, zeros):
    return pl.kernel(
        _deg_body,
        out_type=jax.ShapeDtypeStruct((NC, N, DW), jnp.float32),
        mesh=_mesh,
        compiler_params=pltpu.CompilerParams(use_tc_tiling_on_sc=False),
        scratch_types=[
            pltpu.VMEM((NCHUNK, CH), jnp.int32),
            pltpu.VMEM((CH, DW), jnp.float32),
            pltpu.VMEM_SHARED((NP_, DW), jnp.float32),
            pltpu.SemaphoreType.DMA((4,)),
        ],
    )(dst3, ones, zeros)


def _make_push_body(nb, ga):
    ia = ga + 1
    sl = nb - ga

    def _push_body(zs_hbm, src3_hbm, dst3_hbm, zeros_hbm, part_hbm,
                   sring, dring, rows, acc, ssm, dsm, gsem, ssem):
        c = lax.axis_index("c")
        s = lax.axis_index("s")
        w = c * NS + s

        pltpu.sync_copy(zeros_hbm, acc.at[pl.ds(s * NPS, NPS)])

        for jp in range(ia):
            pltpu.async_copy(src3_hbm.at[w, jp], sring.at[jp], ssm.at[jp])
            pltpu.async_copy(dst3_hbm.at[w, jp], dring.at[jp], dsm.at[jp])

        plsc.subcore_barrier()

        def step(j, carry):
            b = j & (nb - 1)

            @pl.when(j >= sl)
            def _():
                bd = (j - sl) & (nb - 1)
                pltpu.make_async_copy(rows.at[bd],
                                      acc.at[dring.at[(j - sl) & (NI - 1)]],
                                      ssem.at[bd]).wait()

            @pl.when(j + ia < NCHUNK)
            def _():
                i8 = (j + ia) & (NI - 1)
                pltpu.async_copy(src3_hbm.at[w, j + ia], sring.at[i8],
                                 ssm.at[i8])
                pltpu.async_copy(dst3_hbm.at[w, j + ia], dring.at[i8],
                                 dsm.at[i8])

            @pl.when(j + ga < NCHUNK)
            def _():
                i8 = (j + ga) & (NI - 1)
                bg = (j + ga) & (nb - 1)
                pltpu.make_async_copy(src3_hbm.at[w, j + ga], sring.at[i8],
                                      ssm.at[i8]).wait()
                pltpu.async_copy(zs_hbm.at[sring.at[i8]], rows.at[bg],
                                 gsem.at[bg])

            pltpu.make_async_copy(zs_hbm.at[sring.at[j & (NI - 1)]],
                                  rows.at[b], gsem.at[b]).wait()
            pltpu.make_async_copy(dst3_hbm.at[w, j],
                                  dring.at[j & (NI - 1)],
                                  dsm.at[j & (NI - 1)]).wait()
            pltpu.async_copy(rows.at[b], acc.at[dring.at[j & (NI - 1)]],
                             ssem.at[b], add=True)
            return carry

        for jp in range(ga):
            pltpu.make_async_copy(src3_hbm.at[w, jp], sring.at[jp],
                                  ssm.at[jp]).wait()
            pltpu.async_copy(zs_hbm.at[sring.at[jp]], rows.at[jp],
                             gsem.at[jp])

        lax.fori_loop(0, NCHUNK, step, 0)

        for jt in range(NCHUNK - sl, NCHUNK):
            bt = jt & (nb - 1)
            pltpu.make_async_copy(rows.at[bt],
                                  acc.at[dring.at[jt & (NI - 1)]],
                                  ssem.at[bt]).wait()
        plsc.subcore_barrier()

        @pl.when(s < NS - 1)
        def _():
            pltpu.sync_copy(acc.at[pl.ds(s * NPS, NPS)],
                            part_hbm.at[c, pl.ds(s * NPS, NPS)])

        @pl.when(s == NS - 1)
        def _():
            pltpu.sync_copy(acc.at[pl.ds((NS - 1) * NPS, LASTROWS)],
                            part_hbm.at[c, pl.ds((NS - 1) * NPS, LASTROWS)])

    return _push_body


def _sc_push(zs, src3, dst3, zeros):
    d = zs.shape[1]
    nb = 2 if d > 64 else 4
    ga = nb // 2
    return pl.kernel(
        _make_push_body(nb, ga),
        out_type=jax.ShapeDtypeStruct((NC, N, d), jnp.float32),
        mesh=_mesh,
        compiler_params=pltpu.CompilerParams(use_tc_tiling_on_sc=False),
        scratch_types=[
            pltpu.VMEM((NI, CH), jnp.int32),
            pltpu.VMEM((NI, CH), jnp.int32),
            pltpu.VMEM((nb, CH, d), jnp.float32),
            pltpu.VMEM_SHARED((NP_, d), jnp.float32),
            pltpu.SemaphoreType.DMA((NI,)),
            pltpu.SemaphoreType.DMA((NI,)),
            pltpu.SemaphoreType.DMA((nb,)),
            pltpu.SemaphoreType.DMA((nb,)),
        ],
    )(zs, src3, dst3, zeros)


BN = 1000


def _tc_mm_body(x_ref, w_ref, o_ref):
    o_ref[...] = jnp.dot(x_ref[...], w_ref[...],
                         preferred_element_type=jnp.float32)


def _tc_mm(x, W1):
    di = x.shape[1]
    do = W1.shape[1]
    return pl.pallas_call(
        _tc_mm_body,
        grid=(N // BN,),
        in_specs=[
            pl.BlockSpec((BN, di), lambda i: (i, 0)),
            pl.BlockSpec((di, do), lambda i: (0, 0)),
        ],
        out_specs=pl.BlockSpec((BN, do), lambda i: (i, 0)),
        out_shape=jax.ShapeDtypeStruct((N, do), jnp.float32),
    )(x, W1)


def _tc_pre_body(y_ref, d0_ref, d1_ref, o_ref):
    dinv = lax.rsqrt(d0_ref[:, 0:1] + d1_ref[:, 0:1] + 1.0)
    o_ref[...] = dinv * y_ref[...]


def _tc_pre(y1, degf):
    do = y1.shape[1]
    return pl.pallas_call(
        _tc_pre_body,
        grid=(N // BN,),
        in_specs=[
            pl.BlockSpec((BN, do), lambda i: (i, 0)),
            pl.BlockSpec((BN, DW), lambda i: (i, 0)),
            pl.BlockSpec((BN, DW), lambda i: (N // BN + i, 0)),
        ],
        out_specs=pl.BlockSpec((BN, do), lambda i: (i, 0)),
        out_shape=jax.ShapeDtypeStruct((N, do), jnp.float32),
    )(y1, degf, degf)


def _tc_mid_body(p0_ref, p1_ref, zs_ref, d0_ref, d1_ref, b_ref, w_ref, o_ref):
    dinv = lax.rsqrt(d0_ref[:, 0:1] + d1_ref[:, 0:1] + 1.0)
    h = dinv * (p0_ref[...] + p1_ref[...] + zs_ref[...]) + b_ref[...]
    h = jnp.maximum(h, 0.0)
    o_ref[...] = dinv * jnp.dot(h, w_ref[...],
                                preferred_element_type=jnp.float32)


def _tc_mid(p0, p1, zs, degf, b1, W2):
    di = zs.shape[1]
    do = W2.shape[1]
    return pl.pallas_call(
        _tc_mid_body,
        grid=(N // BN,),
        in_specs=[
            pl.BlockSpec((BN, di), lambda i: (i, 0)),
            pl.BlockSpec((BN, di), lambda i: (i, 0)),
            pl.BlockSpec((BN, di), lambda i: (i, 0)),
            pl.BlockSpec((BN, DW), lambda i: (i, 0)),
            pl.BlockSpec((BN, DW), lambda i: (N // BN + i, 0)),
            pl.BlockSpec((1, di), lambda i: (0, 0)),
            pl.BlockSpec((di, do), lambda i: (0, 0)),
        ],
        out_specs=pl.BlockSpec((BN, do), lambda i: (i, 0)),
        out_shape=jax.ShapeDtypeStruct((N, do), jnp.float32),
    )(p0, p1, zs, degf, degf, b1, W2)


def _tc_fin_body(q0_ref, q1_ref, zs_ref, d0_ref, d1_ref, b_ref, wf_ref,
                 bf_ref, pool_ref, out_ref):
    i = pl.program_id(0)
    dinv = lax.rsqrt(d0_ref[:, 0:1] + d1_ref[:, 0:1] + 1.0)
    h = dinv * (q0_ref[...] + q1_ref[...] + zs_ref[...]) + b_ref[...]
    h = jnp.maximum(h, 0.0)

    @pl.when(i == 0)
    def _():
        pool_ref[...] = jnp.zeros_like(pool_ref)

    pool_ref[...] += jnp.sum(h, axis=0, keepdims=True)

    @pl.when(i == pl.num_programs(0) - 1)
    def _():
        out_ref[...] = (jnp.sum(pool_ref[...] * wf_ref[...], axis=1,
                                keepdims=True) / N + bf_ref[...])


def _tc_fin(q0, q1, zs, degf, b2, wfT, bf2):
    di = zs.shape[1]
    _, out = pl.pallas_call(
        _tc_fin_body,
        grid=(N // BN,),
        in_specs=[
            pl.BlockSpec((BN, di), lambda i: (i, 0)),
            pl.BlockSpec((BN, di), lambda i: (i, 0)),
            pl.BlockSpec((BN, di), lambda i: (i, 0)),
            pl.BlockSpec((BN, DW), lambda i: (i, 0)),
            pl.BlockSpec((BN, DW), lambda i: (N // BN + i, 0)),
            pl.BlockSpec((1, di), lambda i: (0, 0)),
            pl.BlockSpec((1, di), lambda i: (0, 0)),
            pl.BlockSpec((1, 1), lambda i: (0, 0)),
        ],
        out_specs=[
            pl.BlockSpec((1, di), lambda i: (0, 0)),
            pl.BlockSpec((1, 1), lambda i: (0, 0)),
        ],
        out_shape=[
            jax.ShapeDtypeStruct((1, di), jnp.float32),
            jax.ShapeDtypeStruct((1, 1), jnp.float32),
        ],
    )(q0, q1, zs, degf, degf, b2, wfT, bf2)
    return out


def kernel(x, edge_index, W1, b1, W2, b2, Wf, bf):
    src3 = edge_index[0].reshape(NW, NCHUNK, CH)
    dst3 = edge_index[1].reshape(NW, NCHUNK, CH)

    ones_deg = jnp.ones((CH, DW), jnp.float32)
    zeros_deg = jnp.zeros((NPS, DW), jnp.float32)
    zeros128 = jnp.zeros((NPS, 128), jnp.float32)
    zeros64 = jnp.zeros((NPS, 64), jnp.float32)

    degp = _sc_degree(dst3, ones_deg, zeros_deg)
    y1 = _tc_mm(x, W1)
    degf = degp.reshape(NC * N, DW)

    zs1 = _tc_pre(y1, degf)
    p = _sc_push(zs1, src3, dst3, zeros128)
    zs2 = _tc_mid(p[0], p[1], zs1, degf, b1.reshape(1, -1), W2)
    q = _sc_push(zs2, src3, dst3, zeros64)
    out = _tc_fin(q[0], q[1], zs2, degf, b2.reshape(1, -1),
                  Wf.reshape(1, -1), bf.reshape(1, 1))
    return out.reshape(1)


# speedup vs baseline: 1.0274x; 1.0274x over previous
import jax
import jax.numpy as jnp
from jax import lax
from jax.experimental import pallas as pl
from jax.experimental.pallas import tpu as pltpu
from jax.experimental.pallas import tpu_sc as plsc

N = 10000
E = 320000
NC = 2
NS = 16
NW = NC * NS
EPT = E // NW
CH = 80
NCHUNK = EPT // CH
NP_ = 10112
NPS = NP_ // NS
LASTROWS = N - (NS - 1) * NPS
DW = 8
NB = 4
NI = 8

_mesh = plsc.VectorSubcoreMesh(core_axis_name="c", subcore_axis_name="s")


def _deg_body(dst3_hbm, ones_hbm, zeros_hbm, deg_hbm, dstbuf, ones_v, acc, sem4):
    c = lax.axis_index("c")
    s = lax.axis_index("s")
    w = c * NS + s

    pltpu.sync_copy(dst3_hbm.at[w], dstbuf)
    pltpu.sync_copy(ones_hbm, ones_v)
    pltpu.sync_copy(zeros_hbm, acc.at[pl.ds(s * NPS, NPS)])
    plsc.subcore_barrier()

    def step(j, carry):
        @pl.when(j >= 4)
        def _():
            pltpu.make_async_copy(ones_v, acc.at[dstbuf.at[j - 4]],
                                  sem4.at[j & 3]).wait()

        pltpu.async_copy(ones_v, acc.at[dstbuf.at[j]], sem4.at[j & 3],
                         add=True)
        return carry

    lax.fori_loop(0, NCHUNK, step, 0)

    for jt in range(NCHUNK - 4, NCHUNK):
        pltpu.make_async_copy(ones_v, acc.at[dstbuf.at[jt]],
                              sem4.at[jt & 3]).wait()

    plsc.subcore_barrier()

    @pl.when(s < NS - 1)
    def _():
        pltpu.sync_copy(acc.at[pl.ds(s * NPS, NPS)],
                        deg_hbm.at[c, pl.ds(s * NPS, NPS)])

    @pl.when(s == NS - 1)
    def _():
        pltpu.sync_copy(acc.at[pl.ds((NS - 1) * NPS, LASTROWS)],
                        deg_hbm.at[c, pl.ds((NS - 1) * NPS, LASTROWS)])


def _sc_degree(dst3, ones, zeros):
    return pl.kernel(
        _deg_body,
        out_type=jax.ShapeDtypeStruct((NC, N, DW), jnp.float32),
        mesh=_mesh,
        compiler_params=pltpu.CompilerParams(use_tc_tiling_on_sc=False),
        scratch_types=[
            pltpu.VMEM((NCHUNK, CH), jnp.int32),
            pltpu.VMEM((CH, DW), jnp.float32),
            pltpu.VMEM_SHARED((NP_, DW), jnp.float32),
            pltpu.SemaphoreType.DMA((4,)),
        ],
    )(dst3, ones, zeros)


def _push_body(zs_hbm, src3_hbm, dst3_hbm, zeros_hbm, part_hbm,
               sring, dring, rows, acc, ssm, dsm, gsem, ssem):
    c = lax.axis_index("c")
    s = lax.axis_index("s")
    w = c * NS + s

    pltpu.sync_copy(zeros_hbm, acc.at[pl.ds(s * NPS, NPS)])

    for jp in range(3):
        pltpu.async_copy(src3_hbm.at[w, jp], sring.at[jp], ssm.at[jp])
        pltpu.async_copy(dst3_hbm.at[w, jp], dring.at[jp], dsm.at[jp])

    plsc.subcore_barrier()

    def step(j, carry):
        b = j & (NB - 1)

        @pl.when(j >= 2)
        def _():
            bd = (j - 2) & (NB - 1)
            pltpu.make_async_copy(rows.at[bd],
                                  acc.at[dring.at[(j - 2) & (NI - 1)]],
                                  ssem.at[bd]).wait()

        @pl.when(j + 3 < NCHUNK)
        def _():
            i8 = (j + 3) & (NI - 1)
            pltpu.async_copy(src3_hbm.at[w, j + 3], sring.at[i8],
                             ssm.at[i8])
            pltpu.async_copy(dst3_hbm.at[w, j + 3], dring.at[i8],
                             dsm.at[i8])

        @pl.when(j + 2 < NCHUNK)
        def _():
            i8 = (j + 2) & (NI - 1)
            bg = (j + 2) & (NB - 1)
            pltpu.make_async_copy(src3_hbm.at[w, j + 2], sring.at[i8],
                                  ssm.at[i8]).wait()
            pltpu.async_copy(zs_hbm.at[sring.at[i8]], rows.at[bg],
                             gsem.at[bg])

        pltpu.make_async_copy(zs_hbm.at[sring.at[j & (NI - 1)]],
                              rows.at[b], gsem.at[b]).wait()
        pltpu.make_async_copy(dst3_hbm.at[w, j], dring.at[j & (NI - 1)],
                              dsm.at[j & (NI - 1)]).wait()
        pltpu.async_copy(rows.at[b], acc.at[dring.at[j & (NI - 1)]],
                         ssem.at[b], add=True)
        return carry

    for jp in range(2):
        pltpu.make_async_copy(src3_hbm.at[w, jp], sring.at[jp],
                              ssm.at[jp]).wait()
        pltpu.async_copy(zs_hbm.at[sring.at[jp]], rows.at[jp],
                         gsem.at[jp])

    lax.fori_loop(0, NCHUNK, step, 0)

    for jt in range(NCHUNK - 2, NCHUNK):
        bt = jt & (NB - 1)
        pltpu.make_async_copy(rows.at[bt],
                              acc.at[dring.at[jt & (NI - 1)]],
                              ssem.at[bt]).wait()
    plsc.subcore_barrier()

    @pl.when(s < NS - 1)
    def _():
        pltpu.sync_copy(acc.at[pl.ds(s * NPS, NPS)],
                        part_hbm.at[c, pl.ds(s * NPS, NPS)])

    @pl.when(s == NS - 1)
    def _():
        pltpu.sync_copy(acc.at[pl.ds((NS - 1) * NPS, LASTROWS)],
                        part_hbm.at[c, pl.ds((NS - 1) * NPS, LASTROWS)])


def _sc_push(zs, src3, dst3, zeros):
    d = zs.shape[1]
    return pl.kernel(
        _push_body,
        out_type=jax.ShapeDtypeStruct((NC, N, d), jnp.float32),
        mesh=_mesh,
        compiler_params=pltpu.CompilerParams(use_tc_tiling_on_sc=False),
        scratch_types=[
            pltpu.VMEM((NI, CH), jnp.int32),
            pltpu.VMEM((NI, CH), jnp.int32),
            pltpu.VMEM((NB, CH, d), jnp.float32),
            pltpu.VMEM_SHARED((NP_, d), jnp.float32),
            pltpu.SemaphoreType.DMA((NI,)),
            pltpu.SemaphoreType.DMA((NI,)),
            pltpu.SemaphoreType.DMA((NB,)),
            pltpu.SemaphoreType.DMA((NB,)),
        ],
    )(zs, src3, dst3, zeros)


BN = 1000


def _tc_mm_body(x_ref, w_ref, o_ref):
    o_ref[...] = jnp.dot(x_ref[...], w_ref[...],
                         preferred_element_type=jnp.float32)


def _tc_mm(x, W1):
    di = x.shape[1]
    do = W1.shape[1]
    return pl.pallas_call(
        _tc_mm_body,
        grid=(N // BN,),
        in_specs=[
            pl.BlockSpec((BN, di), lambda i: (i, 0)),
            pl.BlockSpec((di, do), lambda i: (0, 0)),
        ],
        out_specs=pl.BlockSpec((BN, do), lambda i: (i, 0)),
        out_shape=jax.ShapeDtypeStruct((N, do), jnp.float32),
    )(x, W1)


def _tc_pre_body(y_ref, d0_ref, d1_ref, o_ref):
    dinv = lax.rsqrt(d0_ref[:, 0:1] + d1_ref[:, 0:1] + 1.0)
    o_ref[...] = dinv * y_ref[...]


def _tc_pre(y1, degf):
    do = y1.shape[1]
    return pl.pallas_call(
        _tc_pre_body,
        grid=(N // BN,),
        in_specs=[
            pl.BlockSpec((BN, do), lambda i: (i, 0)),
            pl.BlockSpec((BN, DW), lambda i: (i, 0)),
            pl.BlockSpec((BN, DW), lambda i: (N // BN + i, 0)),
        ],
        out_specs=pl.BlockSpec((BN, do), lambda i: (i, 0)),
        out_shape=jax.ShapeDtypeStruct((N, do), jnp.float32),
    )(y1, degf, degf)


def _tc_mid_body(p0_ref, p1_ref, zs_ref, d0_ref, d1_ref, b_ref, w_ref, o_ref):
    dinv = lax.rsqrt(d0_ref[:, 0:1] + d1_ref[:, 0:1] + 1.0)
    h = dinv * (p0_ref[...] + p1_ref[...] + zs_ref[...]) + b_ref[...]
    h = jnp.maximum(h, 0.0)
    o_ref[...] = dinv * jnp.dot(h, w_ref[...],
                                preferred_element_type=jnp.float32)


def _tc_mid(p0, p1, zs, degf, b1, W2):
    di = zs.shape[1]
    do = W2.shape[1]
    return pl.pallas_call(
        _tc_mid_body,
        grid=(N // BN,),
        in_specs=[
            pl.BlockSpec((BN, di), lambda i: (i, 0)),
            pl.BlockSpec((BN, di), lambda i: (i, 0)),
            pl.BlockSpec((BN, di), lambda i: (i, 0)),
            pl.BlockSpec((BN, DW), lambda i: (i, 0)),
            pl.BlockSpec((BN, DW), lambda i: (N // BN + i, 0)),
            pl.BlockSpec((1, di), lambda i: (0, 0)),
            pl.BlockSpec((di, do), lambda i: (0, 0)),
        ],
        out_specs=pl.BlockSpec((BN, do), lambda i: (i, 0)),
        out_shape=jax.ShapeDtypeStruct((N, do), jnp.float32),
    )(p0, p1, zs, degf, degf, b1, W2)


def _tc_fin_body(q0_ref, q1_ref, zs_ref, d0_ref, d1_ref, b_ref, wf_ref,
                 bf_ref, pool_ref, out_ref):
    i = pl.program_id(0)
    dinv = lax.rsqrt(d0_ref[:, 0:1] + d1_ref[:, 0:1] + 1.0)
    h = dinv * (q0_ref[...] + q1_ref[...] + zs_ref[...]) + b_ref[...]
    h = jnp.maximum(h, 0.0)

    @pl.when(i == 0)
    def _():
        pool_ref[...] = jnp.zeros_like(pool_ref)

    pool_ref[...] += jnp.sum(h, axis=0, keepdims=True)

    @pl.when(i == pl.num_programs(0) - 1)
    def _():
        out_ref[...] = (jnp.sum(pool_ref[...] * wf_ref[...], axis=1,
                                keepdims=True) / N + bf_ref[...])


def _tc_fin(q0, q1, zs, degf, b2, wfT, bf2):
    di = zs.shape[1]
    _, out = pl.pallas_call(
        _tc_fin_body,
        grid=(N // BN,),
        in_specs=[
            pl.BlockSpec((BN, di), lambda i: (i, 0)),
            pl.BlockSpec((BN, di), lambda i: (i, 0)),
            pl.BlockSpec((BN, di), lambda i: (i, 0)),
            pl.BlockSpec((BN, DW), lambda i: (i, 0)),
            pl.BlockSpec((BN, DW), lambda i: (N // BN + i, 0)),
            pl.BlockSpec((1, di), lambda i: (0, 0)),
            pl.BlockSpec((1, di), lambda i: (0, 0)),
            pl.BlockSpec((1, 1), lambda i: (0, 0)),
        ],
        out_specs=[
            pl.BlockSpec((1, di), lambda i: (0, 0)),
            pl.BlockSpec((1, 1), lambda i: (0, 0)),
        ],
        out_shape=[
            jax.ShapeDtypeStruct((1, di), jnp.float32),
            jax.ShapeDtypeStruct((1, 1), jnp.float32),
        ],
    )(q0, q1, zs, degf, degf, b2, wfT, bf2)
    return out


def kernel(x, edge_index, W1, b1, W2, b2, Wf, bf):
    src3 = edge_index[0].reshape(NW, NCHUNK, CH)
    dst3 = edge_index[1].reshape(NW, NCHUNK, CH)

    ones_deg = jnp.ones((CH, DW), jnp.float32)
    zeros_deg = jnp.zeros((NPS, DW), jnp.float32)
    zeros128 = jnp.zeros((NPS, 128), jnp.float32)
    zeros64 = jnp.zeros((NPS, 64), jnp.float32)

    degp = _sc_degree(dst3, ones_deg, zeros_deg)
    y1 = _tc_mm(x, W1)
    degf = degp.reshape(NC * N, DW)

    zs1 = _tc_pre(y1, degf)
    p = _sc_push(zs1, src3, dst3, zeros128)
    zs2 = _tc_mid(p[0], p[1], zs1, degf, b1.reshape(1, -1), W2)
    q = _sc_push(zs2, src3, dst3, zeros64)
    out = _tc_fin(q[0], q[1], zs2, degf, b2.reshape(1, -1),
                  Wf.reshape(1, -1), bf.reshape(1, 1))
    return out.reshape(1)
